# baseline (speedup 1.0000x reference)
import functools

import jax
import jax.numpy as jnp
import numpy as np
from jax import lax
from jax.experimental import pallas as pl
from jax.experimental.pallas import tpu as pltpu
from jax.experimental.pallas import tpu_sc as plsc

N = 10000
E = 320000
NPAD = 10240
CH = 40
NTILES = 32
EPT = E // NTILES
NCH = EPT // CH
HALVES = ((0, 124), (124, 126))
ZROWS = NPAD // 16

_j = np.arange(128)
SEL1 = np.zeros((128, 16), np.float32)
SEL1[_j, _j // 16] = 1.0
E1P = np.zeros((16, 128), np.float32)
E1P[_j // 16, _j] = 1.0
_o = np.arange(64)
SEL2 = np.zeros((64, 16), np.float32)
SEL2[_o, 0] = 1.0
E2P = np.zeros((16, 64), np.float32)
E2P[0, _o] = 1.0


def _splat(v, lane):
    idx = jnp.full((16, 1), lane, jnp.int32)
    dn = lax.GatherDimensionNumbers(
        offset_dims=(), collapsed_slice_dims=(0,), start_index_map=(0,))
    return lax.gather(v, idx, dn, (1,),
                      mode=lax.GatherScatterMode.PROMISE_IN_BOUNDS)


def _make_edge_kernel(D, lane_of_group, unroll):
    ROW = 16 + D
    G = D // 16
    uniq_lanes = sorted(set(lane_of_group))
    mesh = plsc.VectorSubcoreMesh(core_axis_name="c", subcore_axis_name="s")

    @functools.partial(
        pl.kernel,
        mesh=mesh,
        compiler_params=pltpu.CompilerParams(use_tc_tiling_on_sc=False),
        out_type=jax.ShapeDtypeStruct((2, NPAD, ROW), jnp.float32),
        scratch_types=[
            pltpu.VMEM((128, CH), jnp.int32),
            pltpu.VMEM((128, CH), jnp.int32),
            pltpu.VMEM((CH, ROW), jnp.float32),
            pltpu.VMEM((CH, 16), jnp.float32),
            pltpu.VMEM((CH, ROW), jnp.float32),
            pltpu.VMEM((CH, ROW), jnp.float32),
            pltpu.VMEM((CH, 16), jnp.float32),
            pltpu.VMEM((CH, ROW), jnp.float32),
            pltpu.VMEM_SHARED((NPAD, ROW), jnp.float32),
            pltpu.SemaphoreType.DMA,
            pltpu.SemaphoreType.DMA,
            pltpu.SemaphoreType.DMA,
            pltpu.SemaphoreType.DMA,
            pltpu.SemaphoreType.DMA,
            pltpu.SemaphoreType.DMA,
            pltpu.SemaphoreType.DMA,
        ],
    )
    def ek(t_hbm, adst_hbm, src2d_hbm, dst2d_hbm, out_hbm,
           srcbig, dstbig, hv0, adv0, obuf0, hv1, adv1, obuf1,
           accum, sgt0, sga0, sgt1, sga1, ss0, ss1, si):
        cid = lax.axis_index("c")
        sid = lax.axis_index("s")
        tid = cid * 16 + sid
        rowbase = tid * NCH
        zero16 = jnp.zeros((16,), jnp.float32)
        zero16i = jnp.zeros((16,), jnp.int32)

        def zrow(k, carry):
            for j in range(ROW // 16):
                obuf0[k, pl.ds(j * 16, 16)] = zero16
            return carry

        lax.fori_loop(0, CH, zrow, 0)
        zcopies = [
            pltpu.async_copy(
                obuf0, accum.at[pl.ds(sid * ZROWS + rep * CH, CH)], si)
            for rep in range(ZROWS // CH)
        ]
        for c in zcopies:
            c.wait()

        def fire(ci, hv, adv, st, sa):
            g0 = pltpu.async_copy(t_hbm.at[srcbig.at[ci]], hv, st)
            g1 = pltpu.async_copy(adst_hbm.at[dstbig.at[ci]], adv, sa)
            return g0, g1

        def drain_gathers(ci, hv, adv, st, sa):
            pltpu.make_async_copy(t_hbm.at[srcbig.at[ci]], hv, st).wait()
            pltpu.make_async_copy(adst_hbm.at[dstbig.at[ci]], adv, sa).wait()

        def drain_scatter(ci, obuf, sem):
            pltpu.make_async_copy(obuf, accum.at[dstbig.at[ci]], sem).wait()

        def compute(hv, adv, obuf):
            def edge(kk, carry):
                ks = [kk * unroll + u for u in range(unroll)]
                ws = []
                for k in ks:
                    e = hv[k, pl.ds(0, 16)] + adv[k, :]
                    w = jnp.exp(jnp.maximum(e, 0.2 * e))
                    obuf[k, pl.ds(0, 16)] = w
                    ws.append(w)
                for k, w in zip(ks, ws):
                    spl = {l: _splat(w, l) for l in uniq_lanes}
                    for g in range(G):
                        off = 16 * (1 + g)
                        obuf[k, pl.ds(off, 16)] = (
                            spl[lane_of_group[g]] * hv[k, pl.ds(off, 16)])
                return carry

            lax.fori_loop(0, CH // unroll, edge, 0)

        plsc.subcore_barrier()

        for off, M in HALVES:
            ixa = pltpu.async_copy(src2d_hbm.at[pl.ds(rowbase + off, M)],
                                   srcbig.at[pl.ds(0, M)], si)
            ixb = pltpu.async_copy(dst2d_hbm.at[pl.ds(rowbase + off, M)],
                                   dstbig.at[pl.ds(0, M)], si)
            ixa.wait()
            ixb.wait()
            for col in (0, 16, CH - 16):
                srcbig[M, pl.ds(col, 16)] = zero16i
                dstbig[M, pl.ds(col, 16)] = zero16i

            fire(0, hv0, adv0, sgt0, sga0)

            def pair(p, carry):
                ca = 2 * p
                fire(ca + 1, hv1, adv1, sgt1, sga1)
                drain_gathers(ca, hv0, adv0, sgt0, sga0)

                @pl.when(p > 0)
                def _():
                    drain_scatter(ca - 2, obuf0, ss0)

                compute(hv0, adv0, obuf0)
                pltpu.async_copy(obuf0, accum.at[dstbig.at[ca]], ss0,
                                 add=True)
                fire(ca + 2, hv0, adv0, sgt0, sga0)
                drain_gathers(ca + 1, hv1, adv1, sgt1, sga1)

                @pl.when(p > 0)
                def _():
                    drain_scatter(ca - 1, obuf1, ss1)

                compute(hv1, adv1, obuf1)
                pltpu.async_copy(obuf1, accum.at[dstbig.at[ca + 1]], ss1,
                                 add=True)
                return carry

            lax.fori_loop(0, M // 2, pair, 0)
            drain_gathers(M, hv0, adv0, sgt0, sga0)
            drain_scatter(M - 2, obuf0, ss0)
            drain_scatter(M - 1, obuf1, ss1)

        plsc.subcore_barrier()
        pltpu.sync_copy(accum.at[pl.ds(sid * ZROWS, ZROWS)],
                        out_hbm.at[cid, pl.ds(sid * ZROWS, ZROWS)])

    return ek


_edge_k1 = _make_edge_kernel(128, list(range(8)), unroll=8)
_edge_k2 = _make_edge_kernel(64, [0, 0, 0, 0], unroll=8)

_B = 1000


def _proj_kernel(x_ref, w_ref, as_ref, ad_ref, t_ref, adp_ref):
    h = jnp.dot(x_ref[...], w_ref[...], preferred_element_type=jnp.float32)
    asp = jnp.dot(h, as_ref[...], preferred_element_type=jnp.float32)
    t_ref[...] = jnp.concatenate([asp, h], axis=1)
    adp_ref[...] = jnp.dot(h, ad_ref[...], preferred_element_type=jnp.float32)


def _merge1_kernel(p_ref, t_ref, adp_ref, b1_ref, w2_ref,
                   as2_ref, ad2_ref, e1p_ref, t2_ref, adp2_ref):
    p = p_ref[0] + p_ref[1]
    asp = t_ref[:, :16]
    h1 = t_ref[:, 16:]
    asum = asp + adp_ref[...]
    ws = jnp.exp(jnp.maximum(asum, 0.2 * asum))
    wsrep = jnp.dot(ws, e1p_ref[...], preferred_element_type=jnp.float32)
    den = (jnp.dot(p[:, :16] + ws, e1p_ref[...],
                   preferred_element_type=jnp.float32) + 1e-16)
    acc = p[:, 16:] + wsrep * h1
    x2 = jnp.maximum(acc / den + b1_ref[...], 0.0)
    h2 = jnp.dot(x2, w2_ref[...], preferred_element_type=jnp.float32)
    asp2 = jnp.dot(h2, as2_ref[...], preferred_element_type=jnp.float32)
    t2_ref[...] = jnp.concatenate([asp2, h2], axis=1)
    adp2_ref[...] = jnp.dot(h2, ad2_ref[...], preferred_element_type=jnp.float32)


def _merge2_kernel(p_ref, t2_ref, adp_ref, b2_ref, e2p_ref, out_ref):
    p = p_ref[0] + p_ref[1]
    asp = t2_ref[:, :16]
    h2 = t2_ref[:, 16:]
    asum = asp + adp_ref[...]
    ws = jnp.exp(jnp.maximum(asum, 0.2 * asum))
    wsrep = jnp.dot(ws, e2p_ref[...], preferred_element_type=jnp.float32)
    den = (jnp.dot(p[:, :16] + ws, e2p_ref[...],
                   preferred_element_type=jnp.float32) + 1e-16)
    acc = p[:, 16:] + wsrep * h2
    out_ref[...] = jnp.maximum(acc / den + b2_ref[...], 0.0)


def _row_spec(d):
    return pl.BlockSpec((_B, d), lambda i: (i, 0))


def _full_spec(s):
    nd = len(s)
    return pl.BlockSpec(s, lambda i, _nd=nd: (0,) * nd)


def kernel(x, edge_index, W1, a_src1, a_dst1, b1, W2, a_src2, a_dst2, b2):
    src2d = edge_index[0].astype(jnp.int32).reshape(E // CH, CH)
    dst2d = edge_index[1].astype(jnp.int32).reshape(E // CH, CH)

    Asrc1 = a_src1.reshape(128)[:, None] * SEL1
    Adst1 = a_dst1.reshape(128)[:, None] * SEL1
    Asrc2 = a_src2.reshape(64)[:, None] * SEL2
    Adst2 = a_dst2.reshape(64)[:, None] * SEL2

    grid = (N // _B,)

    t1, adp1 = pl.pallas_call(
        _proj_kernel,
        grid=grid,
        in_specs=[_row_spec(128), _full_spec((128, 128)),
                  _full_spec((128, 16)), _full_spec((128, 16))],
        out_specs=[_row_spec(144), _row_spec(16)],
        out_shape=[jax.ShapeDtypeStruct((N, 144), jnp.float32),
                   jax.ShapeDtypeStruct((N, 16), jnp.float32)],
    )(x, W1, Asrc1, Adst1)

    p1 = _edge_k1(t1, adp1, src2d, dst2d)

    t2, adp2 = pl.pallas_call(
        _merge1_kernel,
        grid=grid,
        in_specs=[pl.BlockSpec((2, _B, 144), lambda i: (0, i, 0)),
                  _row_spec(144), _row_spec(16),
                  _full_spec((1, 128)), _full_spec((128, 64)),
                  _full_spec((64, 16)), _full_spec((64, 16)),
                  _full_spec((16, 128))],
        out_specs=[_row_spec(80), _row_spec(16)],
        out_shape=[jax.ShapeDtypeStruct((N, 80), jnp.float32),
                   jax.ShapeDtypeStruct((N, 16), jnp.float32)],
    )(p1, t1, adp1, b1.reshape(1, 128), W2, Asrc2, Adst2, jnp.asarray(E1P))

    p2 = _edge_k2(t2, adp2, src2d, dst2d)

    out = pl.pallas_call(
        _merge2_kernel,
        grid=grid,
        in_specs=[pl.BlockSpec((2, _B, 80), lambda i: (0, i, 0)),
                  _row_spec(80), _row_spec(16),
                  _full_spec((1, 64)), _full_spec((16, 64))],
        out_specs=_row_spec(64),
        out_shape=jax.ShapeDtypeStruct((N, 64), jnp.float32),
    )(p2, t2, adp2, b2.reshape(1, 64), jnp.asarray(E2P))

    return out


# speedup vs baseline: 1.0117x; 1.0117x over previous
import functools

import jax
import jax.numpy as jnp
import numpy as np
from jax import lax
from jax.experimental import pallas as pl
from jax.experimental.pallas import tpu as pltpu
from jax.experimental.pallas import tpu_sc as plsc

N = 10000
E = 320000
NPAD = 10240
CH = 40
NTILES = 32
EPT = E // NTILES
NCH = EPT // CH
HALVES = ((0, 124), (124, 126))
ZROWS = NPAD // 16

_j = np.arange(128)
SEL1 = np.zeros((128, 16), np.float32)
SEL1[_j, _j // 16] = 1.0
E1P = np.zeros((16, 128), np.float32)
E1P[_j // 16, _j] = 1.0
_o = np.arange(64)
SEL2 = np.zeros((64, 16), np.float32)
SEL2[_o, 0] = 1.0
E2P = np.zeros((16, 64), np.float32)
E2P[0, _o] = 1.0


def _splat(v, lane):
    idx = jnp.full((16, 1), lane, jnp.int32)
    dn = lax.GatherDimensionNumbers(
        offset_dims=(), collapsed_slice_dims=(0,), start_index_map=(0,))
    return lax.gather(v, idx, dn, (1,),
                      mode=lax.GatherScatterMode.PROMISE_IN_BOUNDS)


def _make_edge_kernel(D, lane_of_group, unroll):
    ROW = 16 + D
    G = D // 16
    uniq_lanes = sorted(set(lane_of_group))
    mesh = plsc.VectorSubcoreMesh(core_axis_name="c", subcore_axis_name="s")

    @functools.partial(
        pl.kernel,
        mesh=mesh,
        compiler_params=pltpu.CompilerParams(use_tc_tiling_on_sc=False),
        out_type=jax.ShapeDtypeStruct((2, NPAD, ROW), jnp.float32),
        scratch_types=[
            pltpu.VMEM((128, CH), jnp.int32),
            pltpu.VMEM((128, CH), jnp.int32),
            pltpu.VMEM((CH, ROW), jnp.float32),
            pltpu.VMEM((CH, 16), jnp.float32),
            pltpu.VMEM((CH, ROW), jnp.float32),
            pltpu.VMEM((CH, ROW), jnp.float32),
            pltpu.VMEM((CH, 16), jnp.float32),
            pltpu.VMEM((CH, ROW), jnp.float32),
            pltpu.VMEM_SHARED((NPAD, ROW), jnp.float32),
            pltpu.SemaphoreType.DMA,
            pltpu.SemaphoreType.DMA,
            pltpu.SemaphoreType.DMA,
            pltpu.SemaphoreType.DMA,
            pltpu.SemaphoreType.DMA,
            pltpu.SemaphoreType.DMA,
            pltpu.SemaphoreType.DMA,
        ],
    )
    def ek(t_hbm, adst_hbm, src2d_hbm, dst2d_hbm, out_hbm,
           srcbig, dstbig, hv0, adv0, obuf0, hv1, adv1, obuf1,
           accum, sgt0, sga0, sgt1, sga1, ss0, ss1, si):
        cid = lax.axis_index("c")
        sid = lax.axis_index("s")
        tid = cid * 16 + sid
        rowbase = tid * NCH
        zero16 = jnp.zeros((16,), jnp.float32)
        zero16i = jnp.zeros((16,), jnp.int32)

        def zrow(k, carry):
            for j in range(ROW // 16):
                obuf0[k, pl.ds(j * 16, 16)] = zero16
            return carry

        lax.fori_loop(0, CH, zrow, 0)
        zcopies = [
            pltpu.async_copy(
                obuf0, accum.at[pl.ds(sid * ZROWS + rep * CH, CH)], si)
            for rep in range(ZROWS // CH)
        ]
        for c in zcopies:
            c.wait()

        def fire(ci, hv, adv, st, sa):
            g0 = pltpu.async_copy(t_hbm.at[srcbig.at[ci]], hv, st)
            g1 = pltpu.async_copy(adst_hbm.at[dstbig.at[ci]], adv, sa)
            return g0, g1

        def drain_gathers(ci, hv, adv, st, sa):
            pltpu.make_async_copy(t_hbm.at[srcbig.at[ci]], hv, st).wait()
            pltpu.make_async_copy(adst_hbm.at[dstbig.at[ci]], adv, sa).wait()

        def drain_scatter(ci, obuf, sem):
            pltpu.make_async_copy(obuf, accum.at[dstbig.at[ci]], sem).wait()

        def compute(hv, adv, obuf):
            def edge(kk, carry):
                ks = [kk * unroll + u for u in range(unroll)]
                ws = []
                for k in ks:
                    e = hv[k, pl.ds(0, 16)] + adv[k, :]
                    w = jnp.exp(jnp.maximum(e, 0.2 * e))
                    obuf[k, pl.ds(0, 16)] = w
                    ws.append(w)
                for k, w in zip(ks, ws):
                    spl = {l: _splat(w, l) for l in uniq_lanes}
                    for g in range(G):
                        off = 16 * (1 + g)
                        obuf[k, pl.ds(off, 16)] = (
                            spl[lane_of_group[g]] * hv[k, pl.ds(off, 16)])
                return carry

            lax.fori_loop(0, CH // unroll, edge, 0)

        plsc.subcore_barrier()

        for off, M in HALVES:
            ixa = pltpu.async_copy(src2d_hbm.at[pl.ds(rowbase + off, M)],
                                   srcbig.at[pl.ds(0, M)], si)
            ixb = pltpu.async_copy(dst2d_hbm.at[pl.ds(rowbase + off, M)],
                                   dstbig.at[pl.ds(0, M)], si)
            ixa.wait()
            ixb.wait()
            for col in (0, 16, CH - 16):
                srcbig[M, pl.ds(col, 16)] = zero16i
                dstbig[M, pl.ds(col, 16)] = zero16i

            fire(0, hv0, adv0, sgt0, sga0)

            def pair(p, carry):
                ca = 2 * p
                fire(ca + 1, hv1, adv1, sgt1, sga1)
                drain_gathers(ca, hv0, adv0, sgt0, sga0)

                @pl.when(p > 0)
                def _():
                    drain_scatter(ca - 2, obuf0, ss0)

                compute(hv0, adv0, obuf0)
                pltpu.async_copy(obuf0, accum.at[dstbig.at[ca]], ss0,
                                 add=True)
                fire(ca + 2, hv0, adv0, sgt0, sga0)
                drain_gathers(ca + 1, hv1, adv1, sgt1, sga1)

                @pl.when(p > 0)
                def _():
                    drain_scatter(ca - 1, obuf1, ss1)

                compute(hv1, adv1, obuf1)
                pltpu.async_copy(obuf1, accum.at[dstbig.at[ca + 1]], ss1,
                                 add=True)
                return carry

            lax.fori_loop(0, M // 2, pair, 0)
            drain_gathers(M, hv0, adv0, sgt0, sga0)
            drain_scatter(M - 2, obuf0, ss0)
            drain_scatter(M - 1, obuf1, ss1)

        plsc.subcore_barrier()
        pltpu.sync_copy(accum.at[pl.ds(sid * ZROWS, ZROWS)],
                        out_hbm.at[cid, pl.ds(sid * ZROWS, ZROWS)])

    return ek


_edge_k1 = _make_edge_kernel(128, list(range(8)), unroll=4)
_edge_k2 = _make_edge_kernel(64, [0, 0, 0, 0], unroll=4)

_B = 1000


def _proj_kernel(x_ref, w_ref, as_ref, ad_ref, t_ref, adp_ref):
    h = jnp.dot(x_ref[...], w_ref[...], preferred_element_type=jnp.float32)
    asp = jnp.dot(h, as_ref[...], preferred_element_type=jnp.float32)
    t_ref[...] = jnp.concatenate([asp, h], axis=1)
    adp_ref[...] = jnp.dot(h, ad_ref[...], preferred_element_type=jnp.float32)


def _merge1_kernel(p_ref, t_ref, adp_ref, b1_ref, w2_ref,
                   as2_ref, ad2_ref, e1p_ref, t2_ref, adp2_ref):
    p = p_ref[0] + p_ref[1]
    asp = t_ref[:, :16]
    h1 = t_ref[:, 16:]
    asum = asp + adp_ref[...]
    ws = jnp.exp(jnp.maximum(asum, 0.2 * asum))
    wsrep = jnp.dot(ws, e1p_ref[...], preferred_element_type=jnp.float32)
    den = (jnp.dot(p[:, :16] + ws, e1p_ref[...],
                   preferred_element_type=jnp.float32) + 1e-16)
    acc = p[:, 16:] + wsrep * h1
    x2 = jnp.maximum(acc / den + b1_ref[...], 0.0)
    h2 = jnp.dot(x2, w2_ref[...], preferred_element_type=jnp.float32)
    asp2 = jnp.dot(h2, as2_ref[...], preferred_element_type=jnp.float32)
    t2_ref[...] = jnp.concatenate([asp2, h2], axis=1)
    adp2_ref[...] = jnp.dot(h2, ad2_ref[...], preferred_element_type=jnp.float32)


def _merge2_kernel(p_ref, t2_ref, adp_ref, b2_ref, e2p_ref, out_ref):
    p = p_ref[0] + p_ref[1]
    asp = t2_ref[:, :16]
    h2 = t2_ref[:, 16:]
    asum = asp + adp_ref[...]
    ws = jnp.exp(jnp.maximum(asum, 0.2 * asum))
    wsrep = jnp.dot(ws, e2p_ref[...], preferred_element_type=jnp.float32)
    den = (jnp.dot(p[:, :16] + ws, e2p_ref[...],
                   preferred_element_type=jnp.float32) + 1e-16)
    acc = p[:, 16:] + wsrep * h2
    out_ref[...] = jnp.maximum(acc / den + b2_ref[...], 0.0)


def _row_spec(d):
    return pl.BlockSpec((_B, d), lambda i: (i, 0))


def _full_spec(s):
    nd = len(s)
    return pl.BlockSpec(s, lambda i, _nd=nd: (0,) * nd)


def kernel(x, edge_index, W1, a_src1, a_dst1, b1, W2, a_src2, a_dst2, b2):
    src2d = edge_index[0].astype(jnp.int32).reshape(E // CH, CH)
    dst2d = edge_index[1].astype(jnp.int32).reshape(E // CH, CH)

    Asrc1 = a_src1.reshape(128)[:, None] * SEL1
    Adst1 = a_dst1.reshape(128)[:, None] * SEL1
    Asrc2 = a_src2.reshape(64)[:, None] * SEL2
    Adst2 = a_dst2.reshape(64)[:, None] * SEL2

    grid = (N // _B,)

    t1, adp1 = pl.pallas_call(
        _proj_kernel,
        grid=grid,
        in_specs=[_row_spec(128), _full_spec((128, 128)),
                  _full_spec((128, 16)), _full_spec((128, 16))],
        out_specs=[_row_spec(144), _row_spec(16)],
        out_shape=[jax.ShapeDtypeStruct((N, 144), jnp.float32),
                   jax.ShapeDtypeStruct((N, 16), jnp.float32)],
    )(x, W1, Asrc1, Adst1)

    p1 = _edge_k1(t1, adp1, src2d, dst2d)

    t2, adp2 = pl.pallas_call(
        _merge1_kernel,
        grid=grid,
        in_specs=[pl.BlockSpec((2, _B, 144), lambda i: (0, i, 0)),
                  _row_spec(144), _row_spec(16),
                  _full_spec((1, 128)), _full_spec((128, 64)),
                  _full_spec((64, 16)), _full_spec((64, 16)),
                  _full_spec((16, 128))],
        out_specs=[_row_spec(80), _row_spec(16)],
        out_shape=[jax.ShapeDtypeStruct((N, 80), jnp.float32),
                   jax.ShapeDtypeStruct((N, 16), jnp.float32)],
    )(p1, t1, adp1, b1.reshape(1, 128), W2, Asrc2, Adst2, jnp.asarray(E1P))

    p2 = _edge_k2(t2, adp2, src2d, dst2d)

    out = pl.pallas_call(
        _merge2_kernel,
        grid=grid,
        in_specs=[pl.BlockSpec((2, _B, 80), lambda i: (0, i, 0)),
                  _row_spec(80), _row_spec(16),
                  _full_spec((1, 64)), _full_spec((16, 64))],
        out_specs=_row_spec(64),
        out_shape=jax.ShapeDtypeStruct((N, 64), jnp.float32),
    )(p2, t2, adp2, b2.reshape(1, 64), jnp.asarray(E2P))

    return out


# speedup vs baseline: 1.0223x; 1.0105x over previous
import functools

import jax
import jax.numpy as jnp
import numpy as np
from jax import lax
from jax.experimental import pallas as pl
from jax.experimental.pallas import tpu as pltpu
from jax.experimental.pallas import tpu_sc as plsc

N = 10000
E = 320000
NPAD = 10240
CH = 40
NTILES = 32
EPT = E // NTILES
NCH = EPT // CH
HALVES = ((0, 124), (124, 126))
ZROWS = NPAD // 16

_j = np.arange(128)
SEL1 = np.zeros((128, 16), np.float32)
SEL1[_j, _j // 16] = 1.0
E1P = np.zeros((16, 128), np.float32)
E1P[_j // 16, _j] = 1.0
_o = np.arange(64)
SEL2 = np.zeros((64, 16), np.float32)
SEL2[_o, 0] = 1.0
E2P = np.zeros((16, 64), np.float32)
E2P[0, _o] = 1.0


def _splat(v, lane):
    idx = jnp.full((16, 1), lane, jnp.int32)
    dn = lax.GatherDimensionNumbers(
        offset_dims=(), collapsed_slice_dims=(0,), start_index_map=(0,))
    return lax.gather(v, idx, dn, (1,),
                      mode=lax.GatherScatterMode.PROMISE_IN_BOUNDS)


def _make_edge_kernel(D, lane_of_group, unroll):
    ROW = 16 + D
    G = D // 16
    uniq_lanes = sorted(set(lane_of_group))
    mesh = plsc.VectorSubcoreMesh(core_axis_name="c", subcore_axis_name="s")

    @functools.partial(
        pl.kernel,
        mesh=mesh,
        compiler_params=pltpu.CompilerParams(use_tc_tiling_on_sc=False),
        out_type=jax.ShapeDtypeStruct((2, NPAD, ROW), jnp.float32),
        scratch_types=[
            pltpu.VMEM((128, CH), jnp.int32),
            pltpu.VMEM((128, CH), jnp.int32),
            pltpu.VMEM((CH, ROW), jnp.float32),
            pltpu.VMEM((CH, 16), jnp.float32),
            pltpu.VMEM((CH, ROW), jnp.float32),
            pltpu.VMEM((CH, ROW), jnp.float32),
            pltpu.VMEM((CH, 16), jnp.float32),
            pltpu.VMEM((CH, ROW), jnp.float32),
            pltpu.VMEM_SHARED((NPAD, ROW), jnp.float32),
            pltpu.SemaphoreType.DMA,
            pltpu.SemaphoreType.DMA,
            pltpu.SemaphoreType.DMA,
            pltpu.SemaphoreType.DMA,
            pltpu.SemaphoreType.DMA,
            pltpu.SemaphoreType.DMA,
            pltpu.SemaphoreType.DMA,
        ],
    )
    def ek(t_hbm, adst_hbm, src2d_hbm, dst2d_hbm, out_hbm,
           srcbig, dstbig, hv0, adv0, obuf0, hv1, adv1, obuf1,
           accum, sgt0, sga0, sgt1, sga1, ss0, ss1, si):
        cid = lax.axis_index("c")
        sid = lax.axis_index("s")
        tid = cid * 16 + sid
        rowbase = tid * NCH
        zero16 = jnp.zeros((16,), jnp.float32)
        zero16i = jnp.zeros((16,), jnp.int32)

        def zrow(k, carry):
            for j in range(ROW // 16):
                obuf0[k, pl.ds(j * 16, 16)] = zero16
            return carry

        lax.fori_loop(0, CH, zrow, 0)
        zcopies = [
            pltpu.async_copy(
                obuf0, accum.at[pl.ds(sid * ZROWS + rep * CH, CH)], si)
            for rep in range(ZROWS // CH)
        ]
        for c in zcopies:
            c.wait()

        def fire(ci, hv, adv, st, sa):
            g0 = pltpu.async_copy(t_hbm.at[srcbig.at[ci]], hv, st)
            g1 = pltpu.async_copy(adst_hbm.at[dstbig.at[ci]], adv, sa)
            return g0, g1

        def drain_gathers(ci, hv, adv, st, sa):
            pltpu.make_async_copy(t_hbm.at[srcbig.at[ci]], hv, st).wait()
            pltpu.make_async_copy(adst_hbm.at[dstbig.at[ci]], adv, sa).wait()

        def drain_scatter(ci, obuf, sem):
            pltpu.make_async_copy(obuf, accum.at[dstbig.at[ci]], sem).wait()

        def compute(hv, adv, obuf):
            def edge(kk, carry):
                ks = [kk * unroll + u for u in range(unroll)]
                ws = []
                for k in ks:
                    e = hv[k, pl.ds(0, 16)] + adv[k, :]
                    w = jnp.exp(jnp.maximum(e, 0.2 * e))
                    obuf[k, pl.ds(0, 16)] = w
                    ws.append(w)
                for k, w in zip(ks, ws):
                    spl = {l: _splat(w, l) for l in uniq_lanes}
                    for g in range(G):
                        off = 16 * (1 + g)
                        obuf[k, pl.ds(off, 16)] = (
                            spl[lane_of_group[g]] * hv[k, pl.ds(off, 16)])
                return carry

            lax.fori_loop(0, CH // unroll, edge, 0)

        plsc.subcore_barrier()

        for off, M in HALVES:
            ixa = pltpu.async_copy(src2d_hbm.at[pl.ds(rowbase + off, M)],
                                   srcbig.at[pl.ds(0, M)], si)
            ixb = pltpu.async_copy(dst2d_hbm.at[pl.ds(rowbase + off, M)],
                                   dstbig.at[pl.ds(0, M)], si)
            ixa.wait()
            ixb.wait()
            for col in (0, 16, CH - 16):
                srcbig[M, pl.ds(col, 16)] = zero16i
                dstbig[M, pl.ds(col, 16)] = zero16i

            fire(0, hv0, adv0, sgt0, sga0)

            def pair(p, carry):
                ca = 2 * p
                fire(ca + 1, hv1, adv1, sgt1, sga1)
                drain_gathers(ca, hv0, adv0, sgt0, sga0)

                @pl.when(p > 0)
                def _():
                    drain_scatter(ca - 2, obuf0, ss0)

                compute(hv0, adv0, obuf0)
                pltpu.async_copy(obuf0, accum.at[dstbig.at[ca]], ss0,
                                 add=True)
                fire(ca + 2, hv0, adv0, sgt0, sga0)
                drain_gathers(ca + 1, hv1, adv1, sgt1, sga1)

                @pl.when(p > 0)
                def _():
                    drain_scatter(ca - 1, obuf1, ss1)

                compute(hv1, adv1, obuf1)
                pltpu.async_copy(obuf1, accum.at[dstbig.at[ca + 1]], ss1,
                                 add=True)
                return carry

            lax.fori_loop(0, M // 2, pair, 0)
            drain_gathers(M, hv0, adv0, sgt0, sga0)
            drain_scatter(M - 2, obuf0, ss0)
            drain_scatter(M - 1, obuf1, ss1)

        plsc.subcore_barrier()
        pltpu.sync_copy(accum.at[pl.ds(sid * ZROWS, ZROWS)],
                        out_hbm.at[cid, pl.ds(sid * ZROWS, ZROWS)])

    return ek


_edge_k1 = _make_edge_kernel(128, list(range(8)), unroll=4)
_edge_k2 = _make_edge_kernel(64, [0, 0, 0, 0], unroll=4)

_B = 2000


def _proj_kernel(x_ref, w_ref, as_ref, ad_ref, t_ref, adp_ref):
    h = jnp.dot(x_ref[...], w_ref[...], preferred_element_type=jnp.float32)
    asp = jnp.dot(h, as_ref[...], preferred_element_type=jnp.float32)
    t_ref[...] = jnp.concatenate([asp, h], axis=1)
    adp_ref[...] = jnp.dot(h, ad_ref[...], preferred_element_type=jnp.float32)


def _merge1_kernel(p_ref, t_ref, adp_ref, b1_ref, w2_ref,
                   as2_ref, ad2_ref, e1p_ref, t2_ref, adp2_ref):
    p = p_ref[0] + p_ref[1]
    asp = t_ref[:, :16]
    h1 = t_ref[:, 16:]
    asum = asp + adp_ref[...]
    ws = jnp.exp(jnp.maximum(asum, 0.2 * asum))
    wsrep = jnp.dot(ws, e1p_ref[...], preferred_element_type=jnp.float32)
    den = (jnp.dot(p[:, :16] + ws, e1p_ref[...],
                   preferred_element_type=jnp.float32) + 1e-16)
    acc = p[:, 16:] + wsrep * h1
    x2 = jnp.maximum(acc / den + b1_ref[...], 0.0)
    h2 = jnp.dot(x2, w2_ref[...], preferred_element_type=jnp.float32)
    asp2 = jnp.dot(h2, as2_ref[...], preferred_element_type=jnp.float32)
    t2_ref[...] = jnp.concatenate([asp2, h2], axis=1)
    adp2_ref[...] = jnp.dot(h2, ad2_ref[...], preferred_element_type=jnp.float32)


def _merge2_kernel(p_ref, t2_ref, adp_ref, b2_ref, e2p_ref, out_ref):
    p = p_ref[0] + p_ref[1]
    asp = t2_ref[:, :16]
    h2 = t2_ref[:, 16:]
    asum = asp + adp_ref[...]
    ws = jnp.exp(jnp.maximum(asum, 0.2 * asum))
    wsrep = jnp.dot(ws, e2p_ref[...], preferred_element_type=jnp.float32)
    den = (jnp.dot(p[:, :16] + ws, e2p_ref[...],
                   preferred_element_type=jnp.float32) + 1e-16)
    acc = p[:, 16:] + wsrep * h2
    out_ref[...] = jnp.maximum(acc / den + b2_ref[...], 0.0)


def _row_spec(d):
    return pl.BlockSpec((_B, d), lambda i: (i, 0))


def _full_spec(s):
    nd = len(s)
    return pl.BlockSpec(s, lambda i, _nd=nd: (0,) * nd)


def kernel(x, edge_index, W1, a_src1, a_dst1, b1, W2, a_src2, a_dst2, b2):
    src2d = edge_index[0].astype(jnp.int32).reshape(E // CH, CH)
    dst2d = edge_index[1].astype(jnp.int32).reshape(E // CH, CH)

    Asrc1 = a_src1.reshape(128)[:, None] * SEL1
    Adst1 = a_dst1.reshape(128)[:, None] * SEL1
    Asrc2 = a_src2.reshape(64)[:, None] * SEL2
    Adst2 = a_dst2.reshape(64)[:, None] * SEL2

    grid = (N // _B,)

    t1, adp1 = pl.pallas_call(
        _proj_kernel,
        grid=grid,
        in_specs=[_row_spec(128), _full_spec((128, 128)),
                  _full_spec((128, 16)), _full_spec((128, 16))],
        out_specs=[_row_spec(144), _row_spec(16)],
        out_shape=[jax.ShapeDtypeStruct((N, 144), jnp.float32),
                   jax.ShapeDtypeStruct((N, 16), jnp.float32)],
    )(x, W1, Asrc1, Adst1)

    p1 = _edge_k1(t1, adp1, src2d, dst2d)

    t2, adp2 = pl.pallas_call(
        _merge1_kernel,
        grid=grid,
        in_specs=[pl.BlockSpec((2, _B, 144), lambda i: (0, i, 0)),
                  _row_spec(144), _row_spec(16),
                  _full_spec((1, 128)), _full_spec((128, 64)),
                  _full_spec((64, 16)), _full_spec((64, 16)),
                  _full_spec((16, 128))],
        out_specs=[_row_spec(80), _row_spec(16)],
        out_shape=[jax.ShapeDtypeStruct((N, 80), jnp.float32),
                   jax.ShapeDtypeStruct((N, 16), jnp.float32)],
    )(p1, t1, adp1, b1.reshape(1, 128), W2, Asrc2, Adst2, jnp.asarray(E1P))

    p2 = _edge_k2(t2, adp2, src2d, dst2d)

    out = pl.pallas_call(
        _merge2_kernel,
        grid=grid,
        in_specs=[pl.BlockSpec((2, _B, 80), lambda i: (0, i, 0)),
                  _row_spec(80), _row_spec(16),
                  _full_spec((1, 64)), _full_spec((16, 64))],
        out_specs=_row_spec(64),
        out_shape=jax.ShapeDtypeStruct((N, 64), jnp.float32),
    )(p2, t2, adp2, b2.reshape(1, 64), jnp.asarray(E2P))

    return out


# speedup vs baseline: 1.0251x; 1.0028x over previous
import functools

import jax
import jax.numpy as jnp
import numpy as np
from jax import lax
from jax.experimental import pallas as pl
from jax.experimental.pallas import tpu as pltpu
from jax.experimental.pallas import tpu_sc as plsc

N = 10000
E = 320000
NPAD = 10240
CH = 40
NTILES = 32
EPT = E // NTILES
NCH = EPT // CH
HALVES = ((0, 124), (124, 126))
ZROWS = NPAD // 16

_j = np.arange(128)
SEL1 = np.zeros((128, 16), np.float32)
SEL1[_j, _j // 16] = 1.0
E1P = np.zeros((16, 128), np.float32)
E1P[_j // 16, _j] = 1.0
_o = np.arange(64)
SEL2 = np.zeros((64, 16), np.float32)
SEL2[_o, 0] = 1.0
E2P = np.zeros((16, 64), np.float32)
E2P[0, _o] = 1.0


def _splat(v, lane):
    idx = jnp.full((16, 1), lane, jnp.int32)
    dn = lax.GatherDimensionNumbers(
        offset_dims=(), collapsed_slice_dims=(0,), start_index_map=(0,))
    return lax.gather(v, idx, dn, (1,),
                      mode=lax.GatherScatterMode.PROMISE_IN_BOUNDS)


def _make_edge_kernel(D, lane_of_group, unroll):
    ROW = 16 + D
    G = D // 16
    uniq_lanes = sorted(set(lane_of_group))
    mesh = plsc.VectorSubcoreMesh(core_axis_name="c", subcore_axis_name="s")

    @functools.partial(
        pl.kernel,
        mesh=mesh,
        compiler_params=pltpu.CompilerParams(use_tc_tiling_on_sc=False),
        out_type=jax.ShapeDtypeStruct((2, NPAD, ROW), jnp.float32),
        scratch_types=[
            pltpu.VMEM((128, CH), jnp.int32),
            pltpu.VMEM((128, CH), jnp.int32),
            pltpu.VMEM((CH, ROW), jnp.float32),
            pltpu.VMEM((CH, 16), jnp.float32),
            pltpu.VMEM((CH, ROW), jnp.float32),
            pltpu.VMEM((CH, ROW), jnp.float32),
            pltpu.VMEM((CH, 16), jnp.float32),
            pltpu.VMEM((CH, ROW), jnp.float32),
            pltpu.VMEM_SHARED((NPAD, ROW), jnp.float32),
            pltpu.SemaphoreType.DMA,
            pltpu.SemaphoreType.DMA,
            pltpu.SemaphoreType.DMA,
            pltpu.SemaphoreType.DMA,
            pltpu.SemaphoreType.DMA,
            pltpu.SemaphoreType.DMA,
            pltpu.SemaphoreType.DMA,
        ],
    )
    def ek(t_hbm, adst_hbm, src2d_hbm, dst2d_hbm, out_hbm,
           srcbig, dstbig, hv0, adv0, obuf0, hv1, adv1, obuf1,
           accum, sgt0, sga0, sgt1, sga1, ss0, ss1, si):
        cid = lax.axis_index("c")
        sid = lax.axis_index("s")
        tid = cid * 16 + sid
        rowbase = tid * NCH
        zero16 = jnp.zeros((16,), jnp.float32)
        zero16i = jnp.zeros((16,), jnp.int32)

        def zrow(k, carry):
            for j in range(ROW // 16):
                obuf0[k, pl.ds(j * 16, 16)] = zero16
            return carry

        lax.fori_loop(0, CH, zrow, 0)
        zcopies = [
            pltpu.async_copy(
                obuf0, accum.at[pl.ds(sid * ZROWS + rep * CH, CH)], si)
            for rep in range(ZROWS // CH)
        ]
        for c in zcopies:
            c.wait()

        def fire(ci, hv, adv, st, sa):
            g0 = pltpu.async_copy(t_hbm.at[srcbig.at[ci]], hv, st)
            g1 = pltpu.async_copy(adst_hbm.at[dstbig.at[ci]], adv, sa)
            return g0, g1

        def drain_gathers(ci, hv, adv, st, sa):
            pltpu.make_async_copy(t_hbm.at[srcbig.at[ci]], hv, st).wait()
            pltpu.make_async_copy(adst_hbm.at[dstbig.at[ci]], adv, sa).wait()

        def drain_scatter(ci, obuf, sem):
            pltpu.make_async_copy(obuf, accum.at[dstbig.at[ci]], sem).wait()

        def compute(hv, adv, obuf):
            def edge(kk, carry):
                ks = [kk * unroll + u for u in range(unroll)]
                ws = []
                for k in ks:
                    e = hv[k, pl.ds(0, 16)] + adv[k, :]
                    w = jnp.exp(jnp.maximum(e, 0.2 * e))
                    obuf[k, pl.ds(0, 16)] = w
                    ws.append(w)
                for k, w in zip(ks, ws):
                    spl = {l: _splat(w, l) for l in uniq_lanes}
                    for g in range(G):
                        off = 16 * (1 + g)
                        obuf[k, pl.ds(off, 16)] = (
                            spl[lane_of_group[g]] * hv[k, pl.ds(off, 16)])
                return carry

            lax.fori_loop(0, CH // unroll, edge, 0)

        plsc.subcore_barrier()

        for off, M in HALVES:
            ixa = pltpu.async_copy(src2d_hbm.at[pl.ds(rowbase + off, M)],
                                   srcbig.at[pl.ds(0, M)], si)
            ixb = pltpu.async_copy(dst2d_hbm.at[pl.ds(rowbase + off, M)],
                                   dstbig.at[pl.ds(0, M)], si)
            ixa.wait()
            ixb.wait()
            for col in (0, 16, CH - 16):
                srcbig[M, pl.ds(col, 16)] = zero16i
                dstbig[M, pl.ds(col, 16)] = zero16i

            fire(0, hv0, adv0, sgt0, sga0)

            def pair(p, carry):
                ca = 2 * p
                fire(ca + 1, hv1, adv1, sgt1, sga1)
                drain_gathers(ca, hv0, adv0, sgt0, sga0)

                @pl.when(p > 0)
                def _():
                    drain_scatter(ca - 2, obuf0, ss0)

                compute(hv0, adv0, obuf0)
                pltpu.async_copy(obuf0, accum.at[dstbig.at[ca]], ss0,
                                 add=True)
                fire(ca + 2, hv0, adv0, sgt0, sga0)
                drain_gathers(ca + 1, hv1, adv1, sgt1, sga1)

                @pl.when(p > 0)
                def _():
                    drain_scatter(ca - 1, obuf1, ss1)

                compute(hv1, adv1, obuf1)
                pltpu.async_copy(obuf1, accum.at[dstbig.at[ca + 1]], ss1,
                                 add=True)
                return carry

            lax.fori_loop(0, M // 2, pair, 0)
            drain_gathers(M, hv0, adv0, sgt0, sga0)
            drain_scatter(M - 2, obuf0, ss0)
            drain_scatter(M - 1, obuf1, ss1)

        plsc.subcore_barrier()
        pltpu.sync_copy(accum.at[pl.ds(sid * ZROWS, ZROWS)],
                        out_hbm.at[cid, pl.ds(sid * ZROWS, ZROWS)])

    return ek


_edge_k1 = _make_edge_kernel(128, list(range(8)), unroll=4)
_edge_k2 = _make_edge_kernel(64, [0, 0, 0, 0], unroll=4)

_B = 5000


def _proj_kernel(x_ref, w_ref, as_ref, ad_ref, t_ref, adp_ref):
    h = jnp.dot(x_ref[...], w_ref[...], preferred_element_type=jnp.float32)
    asp = jnp.dot(h, as_ref[...], preferred_element_type=jnp.float32)
    t_ref[...] = jnp.concatenate([asp, h], axis=1)
    adp_ref[...] = jnp.dot(h, ad_ref[...], preferred_element_type=jnp.float32)


def _merge1_kernel(p_ref, t_ref, adp_ref, b1_ref, w2_ref,
                   as2_ref, ad2_ref, e1p_ref, t2_ref, adp2_ref):
    p = p_ref[0] + p_ref[1]
    asp = t_ref[:, :16]
    h1 = t_ref[:, 16:]
    asum = asp + adp_ref[...]
    ws = jnp.exp(jnp.maximum(asum, 0.2 * asum))
    wsrep = jnp.dot(ws, e1p_ref[...], preferred_element_type=jnp.float32)
    den = (jnp.dot(p[:, :16] + ws, e1p_ref[...],
                   preferred_element_type=jnp.float32) + 1e-16)
    acc = p[:, 16:] + wsrep * h1
    x2 = jnp.maximum(acc / den + b1_ref[...], 0.0)
    h2 = jnp.dot(x2, w2_ref[...], preferred_element_type=jnp.float32)
    asp2 = jnp.dot(h2, as2_ref[...], preferred_element_type=jnp.float32)
    t2_ref[...] = jnp.concatenate([asp2, h2], axis=1)
    adp2_ref[...] = jnp.dot(h2, ad2_ref[...], preferred_element_type=jnp.float32)


def _merge2_kernel(p_ref, t2_ref, adp_ref, b2_ref, e2p_ref, out_ref):
    p = p_ref[0] + p_ref[1]
    asp = t2_ref[:, :16]
    h2 = t2_ref[:, 16:]
    asum = asp + adp_ref[...]
    ws = jnp.exp(jnp.maximum(asum, 0.2 * asum))
    wsrep = jnp.dot(ws, e2p_ref[...], preferred_element_type=jnp.float32)
    den = (jnp.dot(p[:, :16] + ws, e2p_ref[...],
                   preferred_element_type=jnp.float32) + 1e-16)
    acc = p[:, 16:] + wsrep * h2
    out_ref[...] = jnp.maximum(acc / den + b2_ref[...], 0.0)


def _row_spec(d):
    return pl.BlockSpec((_B, d), lambda i: (i, 0))


def _full_spec(s):
    nd = len(s)
    return pl.BlockSpec(s, lambda i, _nd=nd: (0,) * nd)


def kernel(x, edge_index, W1, a_src1, a_dst1, b1, W2, a_src2, a_dst2, b2):
    src2d = edge_index[0].astype(jnp.int32).reshape(E // CH, CH)
    dst2d = edge_index[1].astype(jnp.int32).reshape(E // CH, CH)

    Asrc1 = a_src1.reshape(128)[:, None] * SEL1
    Adst1 = a_dst1.reshape(128)[:, None] * SEL1
    Asrc2 = a_src2.reshape(64)[:, None] * SEL2
    Adst2 = a_dst2.reshape(64)[:, None] * SEL2

    grid = (N // _B,)

    t1, adp1 = pl.pallas_call(
        _proj_kernel,
        grid=grid,
        in_specs=[_row_spec(128), _full_spec((128, 128)),
                  _full_spec((128, 16)), _full_spec((128, 16))],
        out_specs=[_row_spec(144), _row_spec(16)],
        out_shape=[jax.ShapeDtypeStruct((N, 144), jnp.float32),
                   jax.ShapeDtypeStruct((N, 16), jnp.float32)],
    )(x, W1, Asrc1, Adst1)

    p1 = _edge_k1(t1, adp1, src2d, dst2d)

    t2, adp2 = pl.pallas_call(
        _merge1_kernel,
        grid=grid,
        in_specs=[pl.BlockSpec((2, _B, 144), lambda i: (0, i, 0)),
                  _row_spec(144), _row_spec(16),
                  _full_spec((1, 128)), _full_spec((128, 64)),
                  _full_spec((64, 16)), _full_spec((64, 16)),
                  _full_spec((16, 128))],
        out_specs=[_row_spec(80), _row_spec(16)],
        out_shape=[jax.ShapeDtypeStruct((N, 80), jnp.float32),
                   jax.ShapeDtypeStruct((N, 16), jnp.float32)],
    )(p1, t1, adp1, b1.reshape(1, 128), W2, Asrc2, Adst2, jnp.asarray(E1P))

    p2 = _edge_k2(t2, adp2, src2d, dst2d)

    out = pl.pallas_call(
        _merge2_kernel,
        grid=grid,
        in_specs=[pl.BlockSpec((2, _B, 80), lambda i: (0, i, 0)),
                  _row_spec(80), _row_spec(16),
                  _full_spec((1, 64)), _full_spec((16, 64))],
        out_specs=_row_spec(64),
        out_shape=jax.ShapeDtypeStruct((N, 64), jnp.float32),
    )(p2, t2, adp2, b2.reshape(1, 64), jnp.asarray(E2P))

    return out
